# Initial kernel scaffold; baseline (speedup 1.0000x reference)
import jax
import jax.numpy as jnp
from jax.experimental import pallas as pl


def kernel(x, edge_index, fb_w, fb_b, bn_g, bn_b, kan_base_w, kan_spline_w, gl1_w, gl1_b, gl2_w, gl2_b, dec1_w, dec1_b, dec2_w, dec2_b):
    raise NotImplementedError("write your pallas kernel here")


# speedup vs baseline: 3.3102x; 3.3102x over previous
import functools

import jax
import jax.numpy as jnp
import numpy as np
from jax import lax
from jax.experimental import pallas as pl
from jax.experimental.pallas import tpu as pltpu
from jax.experimental.pallas import tpu_sc as plsc

_N = 10000
_E = 320000
_D_IN = 128
_D_HID = 128
_D_OUT = 64
_GRID_SIZE = 5
_SPLINE_ORDER = 3
_NB = _GRID_SIZE + _SPLINE_ORDER

_NPAD = 10240
_EPAD = 327680
_NC, _NS, _NW = 2, 16, 32
_CH = 128
_RB = 1024
_NBLK = _NPAD // _RB

_KNOTS = [float(v) for v in
          (np.arange(-_SPLINE_ORDER, _GRID_SIZE + _SPLINE_ORDER + 1)
           * (2.0 / _GRID_SIZE) - 1.0).astype(np.float32)]


@functools.cache
def _sc_spmm(F):
    epw = _EPAD // _NW
    nch = epw // _CH
    zrows = _NPAD // _NS
    mesh = plsc.VectorSubcoreMesh(core_axis_name="c", subcore_axis_name="s")

    def body(src_hbm, dst_hbm, table_hbm, zeros_hbm, out_hbm,
             src_v, dst_v, rows_v, acc_sh, sem):
        cid = lax.axis_index("c")
        sid = lax.axis_index("s")
        wid = sid * _NC + cid
        pltpu.sync_copy(zeros_hbm.at[pl.ds(sid * zrows, zrows)],
                        acc_sh.at[pl.ds(sid * zrows, zrows)])
        plsc.subcore_barrier()
        base = wid * epw

        def step(i, carry):
            off = base + i * _CH
            pltpu.sync_copy(src_hbm.at[pl.ds(off, _CH)], src_v)
            pltpu.sync_copy(dst_hbm.at[pl.ds(off, _CH)], dst_v)
            pltpu.async_copy(table_hbm.at[src_v], rows_v, sem).wait()
            pltpu.sync_copy(rows_v, acc_sh.at[dst_v], add=True)
            return carry

        lax.fori_loop(0, nch, step, 0)
        plsc.subcore_barrier()
        pltpu.sync_copy(acc_sh.at[pl.ds(sid * zrows, zrows)],
                        out_hbm.at[cid, pl.ds(sid * zrows, zrows)])

    return pl.kernel(
        body,
        out_type=jax.ShapeDtypeStruct((_NC, _NPAD, F), jnp.float32),
        mesh=mesh,
        scratch_types=[
            pltpu.VMEM((_CH,), jnp.int32),
            pltpu.VMEM((_CH,), jnp.int32),
            pltpu.VMEM((_CH, F), jnp.float32),
            pltpu.VMEM_SHARED((_NPAD, F), jnp.float32),
            pltpu.SemaphoreType.DMA,
        ],
        compiler_params=pltpu.CompilerParams(use_tc_tiling_on_sc=False),
    )


def _full(shape):
    return pl.BlockSpec(shape, lambda i: (0,) * len(shape))


def _rows(bs, *rest):
    return pl.BlockSpec((bs,) + rest, lambda i: (i,) + (0,) * len(rest))


def _a_body(x_ref, wt_ref, b_ref, h_ref, st_ref):
    i = pl.program_id(0)
    h = jnp.dot(x_ref[...], wt_ref[...],
                preferred_element_type=jnp.float32) + b_ref[...]
    h_ref[...] = h
    rows = i * _RB + lax.broadcasted_iota(jnp.int32, (_RB, 1), 0)
    hm = jnp.where(rows < _N, h, 0.0)
    s = jnp.concatenate([jnp.sum(hm, 0, keepdims=True),
                         jnp.sum(hm * hm, 0, keepdims=True)], 0)

    @pl.when(i == 0)
    def _():
        st_ref[...] = jnp.zeros_like(st_ref)

    st_ref[...] += s


def _stage_a(xp, fb_wt, fb_b):
    return pl.pallas_call(
        _a_body,
        grid=(_NBLK,),
        in_specs=[_rows(_RB, _D_IN), _full((_D_IN, _D_HID)), _full((1, _D_HID))],
        out_specs=[_rows(_RB, _D_HID), _full((2, _D_HID))],
        out_shape=[jax.ShapeDtypeStruct((_NPAD, _D_HID), jnp.float32),
                   jax.ShapeDtypeStruct((2, _D_HID), jnp.float32)],
    )(xp, fb_wt, fb_b)


def _b_body(h_ref, st_ref, bng_ref, bnb_ref, kbwt_ref, wspt_ref, feat_ref):
    h = h_ref[...]
    mu = st_ref[0:1, :] * (1.0 / _N)
    var = st_ref[1:2, :] * (1.0 / _N) - mu * mu
    hb = bng_ref[...] * (h - mu) / jnp.sqrt(var + 1e-3) + bnb_ref[...]
    hb = jnp.where(hb > 0, hb, jnp.exp(jnp.minimum(hb, 0.0)) - 1.0)
    sil = hb / (1.0 + jnp.exp(-hb))
    bs = [jnp.where((hb >= _KNOTS[t]) & (hb < _KNOTS[t + 1]), 1.0, 0.0)
          for t in range(len(_KNOTS) - 1)]
    for k in range(1, _SPLINE_ORDER + 1):
        nxt = []
        for t in range(len(bs) - 1):
            left = (hb - _KNOTS[t]) / (_KNOTS[t + k] - _KNOTS[t])
            right = (_KNOTS[t + k + 1] - hb) / (_KNOTS[t + k + 1] - _KNOTS[t + 1])
            nxt.append(left * bs[t] + right * bs[t + 1])
        bs = nxt
    acc = jnp.dot(sil, kbwt_ref[...], preferred_element_type=jnp.float32)
    for t in range(_NB):
        acc += jnp.dot(bs[t], wspt_ref[t], preferred_element_type=jnp.float32)
    feat_ref[...] = acc


def _stage_b(h, st, bn_g, bn_b, kbw_t, wsp_t):
    return pl.pallas_call(
        _b_body,
        grid=(_NBLK,),
        in_specs=[_rows(_RB, _D_HID), _full((2, _D_HID)), _full((1, _D_HID)),
                  _full((1, _D_HID)), _full((_D_HID, _D_OUT)),
                  _full((_NB, _D_HID, _D_OUT))],
        out_specs=_rows(_RB, _D_OUT),
        out_shape=jax.ShapeDtypeStruct((_NPAD, _D_OUT), jnp.float32),
    )(h, st, bn_g, bn_b, kbw_t, wsp_t)


def _c_body(p_ref, w1_ref, b1_ref, w2_ref, out_ref):
    g = p_ref[0] + p_ref[1]
    z1 = jnp.maximum(
        jnp.dot(g, w1_ref[...], preferred_element_type=jnp.float32)
        + b1_ref[...], 0.0)
    out_ref[...] = jnp.dot(z1, w2_ref[...], preferred_element_type=jnp.float32)


def _stage_c(p, w1, b1, w2, fin, fmid, fout):
    return pl.pallas_call(
        _c_body,
        grid=(_NBLK,),
        in_specs=[pl.BlockSpec((_NC, _RB, fin), lambda i: (0, i, 0)),
                  _full((fin, fmid)), _full((1, fmid)), _full((fmid, fout))],
        out_specs=_rows(_RB, fout),
        out_shape=jax.ShapeDtypeStruct((_NPAD, fout), jnp.float32),
    )(p, w1, b1, w2)


def _d_body(p_ref, b_ref, out_ref):
    out_ref[...] = p_ref[0] + p_ref[1] + b_ref[...]


def _stage_d(p, b, f):
    return pl.pallas_call(
        _d_body,
        grid=(_NBLK,),
        in_specs=[pl.BlockSpec((_NC, _RB, f), lambda i: (0, i, 0)),
                  _full((1, f))],
        out_specs=_rows(_RB, f),
        out_shape=jax.ShapeDtypeStruct((_NPAD, f), jnp.float32),
    )(p, b)


def kernel(x, edge_index, fb_w, fb_b, bn_g, bn_b, kan_base_w, kan_spline_w,
           gl1_w, gl1_b, gl2_w, gl2_b, dec1_w, dec1_b, dec2_w, dec2_b):
    f32 = jnp.float32
    xp = jnp.zeros((_NPAD, _D_IN), f32).at[:_N].set(x)
    pad = jnp.full((_EPAD - _E,), _N, jnp.int32)
    src_p = jnp.concatenate([edge_index[0], pad])
    dst_p = jnp.concatenate([edge_index[1], pad])
    z64 = jnp.zeros((_NPAD, _D_OUT), f32)
    z128 = jnp.zeros((_NPAD, _D_IN), f32)
    fb_wt = fb_w.T
    kbw_t = kan_base_w.T
    wsp_t = jnp.transpose(kan_spline_w, (2, 1, 0))

    h, st = _stage_a(xp, fb_wt, fb_b.reshape(1, -1))
    feat = _stage_b(h, st, bn_g.reshape(1, -1), bn_b.reshape(1, -1),
                    kbw_t, wsp_t)

    p1 = _sc_spmm(_D_OUT)(src_p, dst_p, feat, z64)
    m2 = _stage_c(p1, gl1_w, gl1_b.reshape(1, -1), gl2_w,
                  _D_OUT, _D_HID, _D_OUT)
    p2 = _sc_spmm(_D_OUT)(src_p, dst_p, m2, z64)
    latent = _stage_d(p2, gl2_b.reshape(1, -1), _D_OUT)
    p3 = _sc_spmm(_D_OUT)(src_p, dst_p, latent, z64)
    m4 = _stage_c(p3, dec1_w, dec1_b.reshape(1, -1), dec2_w,
                  _D_OUT, _D_HID, _D_IN)
    p4 = _sc_spmm(_D_IN)(src_p, dst_p, m4, z128)
    recon = _stage_d(p4, dec2_b.reshape(1, -1), _D_IN)
    return recon[:_N]


# speedup vs baseline: 4.4758x; 1.3521x over previous
import functools

import jax
import jax.numpy as jnp
import numpy as np
from jax import lax
from jax.experimental import pallas as pl
from jax.experimental.pallas import tpu as pltpu
from jax.experimental.pallas import tpu_sc as plsc

_N = 10000
_E = 320000
_D_IN = 128
_D_HID = 128
_D_OUT = 64
_GRID_SIZE = 5
_SPLINE_ORDER = 3
_NB = _GRID_SIZE + _SPLINE_ORDER

_NPAD = 10240
_EPAD = 327680
_NC, _NS, _NW = 2, 16, 32
_CH = 128
_RB = 1024
_NBLK = _NPAD // _RB

_KNOTS = [float(v) for v in
          (np.arange(-_SPLINE_ORDER, _GRID_SIZE + _SPLINE_ORDER + 1)
           * (2.0 / _GRID_SIZE) - 1.0).astype(np.float32)]


@functools.cache
def _sc_spmm(F):
    epw = _EPAD // _NW
    nch = epw // _CH
    zrows = _NPAD // _NS
    mesh = plsc.VectorSubcoreMesh(core_axis_name="c", subcore_axis_name="s")

    def body(src_hbm, dst_hbm, table_hbm, zeros_hbm, out_hbm,
             src_v, dst_v, rows_v, acc_sh, gsem, isem):
        cid = lax.axis_index("c")
        sid = lax.axis_index("s")
        wid = sid * _NC + cid
        iload = pltpu.async_copy(src_hbm.at[pl.ds(wid * nch, nch)], src_v, isem)
        iload2 = pltpu.async_copy(dst_hbm.at[pl.ds(wid * nch, nch)], dst_v, isem)
        pltpu.sync_copy(zeros_hbm.at[pl.ds(sid * zrows, zrows)],
                        acc_sh.at[pl.ds(sid * zrows, zrows)])
        iload.wait()
        iload2.wait()
        plsc.subcore_barrier()

        def gather(i, slot):
            pltpu.async_copy(table_hbm.at[src_v.at[i]], rows_v.at[slot], gsem)

        def wait_gather():
            pltpu.make_async_copy(table_hbm.at[src_v.at[0]],
                                  rows_v.at[0], gsem).wait()

        gather(0, 0)

        def step(i, carry):
            slot = lax.rem(i, 2)

            @pl.when(i + 1 < nch)
            def _():
                gather(i + 1, 1 - slot)

            wait_gather()
            pltpu.sync_copy(rows_v.at[slot], acc_sh.at[dst_v.at[i]], add=True)
            return carry

        lax.fori_loop(0, nch, step, 0)
        plsc.subcore_barrier()
        pltpu.sync_copy(acc_sh.at[pl.ds(sid * zrows, zrows)],
                        out_hbm.at[cid, pl.ds(sid * zrows, zrows)])

    return pl.kernel(
        body,
        out_type=jax.ShapeDtypeStruct((_NC, _NPAD, F), jnp.float32),
        mesh=mesh,
        scratch_types=[
            pltpu.VMEM((nch, _CH), jnp.int32),
            pltpu.VMEM((nch, _CH), jnp.int32),
            pltpu.VMEM((2, _CH, F), jnp.float32),
            pltpu.VMEM_SHARED((_NPAD, F), jnp.float32),
            pltpu.SemaphoreType.DMA,
            pltpu.SemaphoreType.DMA,
        ],
        compiler_params=pltpu.CompilerParams(use_tc_tiling_on_sc=False),
    )


def _full(shape):
    return pl.BlockSpec(shape, lambda i: (0,) * len(shape))


def _rows(bs, *rest):
    return pl.BlockSpec((bs,) + rest, lambda i: (i,) + (0,) * len(rest))


def _a_body(x_ref, wt_ref, b_ref, h_ref, st_ref):
    i = pl.program_id(0)
    h = jnp.dot(x_ref[...], wt_ref[...],
                preferred_element_type=jnp.float32) + b_ref[...]
    h_ref[...] = h
    rows = i * _RB + lax.broadcasted_iota(jnp.int32, (_RB, 1), 0)
    hm = jnp.where(rows < _N, h, 0.0)
    s = jnp.concatenate([jnp.sum(hm, 0, keepdims=True),
                         jnp.sum(hm * hm, 0, keepdims=True)], 0)

    @pl.when(i == 0)
    def _():
        st_ref[...] = jnp.zeros_like(st_ref)

    st_ref[...] += s


def _stage_a(xp, fb_wt, fb_b):
    return pl.pallas_call(
        _a_body,
        grid=(_NBLK,),
        in_specs=[_rows(_RB, _D_IN), _full((_D_IN, _D_HID)), _full((1, _D_HID))],
        out_specs=[_rows(_RB, _D_HID), _full((2, _D_HID))],
        out_shape=[jax.ShapeDtypeStruct((_NPAD, _D_HID), jnp.float32),
                   jax.ShapeDtypeStruct((2, _D_HID), jnp.float32)],
    )(xp, fb_wt, fb_b)


def _b_body(h_ref, st_ref, bng_ref, bnb_ref, kbwt_ref, wspt_ref, feat_ref):
    h = h_ref[...]
    mu = st_ref[0:1, :] * (1.0 / _N)
    var = st_ref[1:2, :] * (1.0 / _N) - mu * mu
    hb = bng_ref[...] * (h - mu) / jnp.sqrt(var + 1e-3) + bnb_ref[...]
    hb = jnp.where(hb > 0, hb, jnp.exp(jnp.minimum(hb, 0.0)) - 1.0)
    sil = hb / (1.0 + jnp.exp(-hb))
    bs = [jnp.where((hb >= _KNOTS[t]) & (hb < _KNOTS[t + 1]), 1.0, 0.0)
          for t in range(len(_KNOTS) - 1)]
    for k in range(1, _SPLINE_ORDER + 1):
        nxt = []
        for t in range(len(bs) - 1):
            left = (hb - _KNOTS[t]) / (_KNOTS[t + k] - _KNOTS[t])
            right = (_KNOTS[t + k + 1] - hb) / (_KNOTS[t + k + 1] - _KNOTS[t + 1])
            nxt.append(left * bs[t] + right * bs[t + 1])
        bs = nxt
    acc = jnp.dot(sil, kbwt_ref[...], preferred_element_type=jnp.float32)
    for t in range(_NB):
        acc += jnp.dot(bs[t], wspt_ref[t], preferred_element_type=jnp.float32)
    feat_ref[...] = acc


def _stage_b(h, st, bn_g, bn_b, kbw_t, wsp_t):
    return pl.pallas_call(
        _b_body,
        grid=(_NBLK,),
        in_specs=[_rows(_RB, _D_HID), _full((2, _D_HID)), _full((1, _D_HID)),
                  _full((1, _D_HID)), _full((_D_HID, _D_OUT)),
                  _full((_NB, _D_HID, _D_OUT))],
        out_specs=_rows(_RB, _D_OUT),
        out_shape=jax.ShapeDtypeStruct((_NPAD, _D_OUT), jnp.float32),
    )(h, st, bn_g, bn_b, kbw_t, wsp_t)


def _c_body(p_ref, w1_ref, b1_ref, w2_ref, out_ref):
    g = p_ref[0] + p_ref[1]
    z1 = jnp.maximum(
        jnp.dot(g, w1_ref[...], preferred_element_type=jnp.float32)
        + b1_ref[...], 0.0)
    out_ref[...] = jnp.dot(z1, w2_ref[...], preferred_element_type=jnp.float32)


def _stage_c(p, w1, b1, w2, fin, fmid, fout):
    return pl.pallas_call(
        _c_body,
        grid=(_NBLK,),
        in_specs=[pl.BlockSpec((_NC, _RB, fin), lambda i: (0, i, 0)),
                  _full((fin, fmid)), _full((1, fmid)), _full((fmid, fout))],
        out_specs=_rows(_RB, fout),
        out_shape=jax.ShapeDtypeStruct((_NPAD, fout), jnp.float32),
    )(p, w1, b1, w2)


def _d_body(p_ref, b_ref, out_ref):
    out_ref[...] = p_ref[0] + p_ref[1] + b_ref[...]


def _stage_d(p, b, f):
    return pl.pallas_call(
        _d_body,
        grid=(_NBLK,),
        in_specs=[pl.BlockSpec((_NC, _RB, f), lambda i: (0, i, 0)),
                  _full((1, f))],
        out_specs=_rows(_RB, f),
        out_shape=jax.ShapeDtypeStruct((_NPAD, f), jnp.float32),
    )(p, b)


def kernel(x, edge_index, fb_w, fb_b, bn_g, bn_b, kan_base_w, kan_spline_w,
           gl1_w, gl1_b, gl2_w, gl2_b, dec1_w, dec1_b, dec2_w, dec2_b):
    f32 = jnp.float32
    xp = jnp.zeros((_NPAD, _D_IN), f32).at[:_N].set(x)
    pad = jnp.full((_EPAD - _E,), _N, jnp.int32)
    src_p = jnp.concatenate([edge_index[0], pad]).reshape(_EPAD // _CH, _CH)
    dst_p = jnp.concatenate([edge_index[1], pad]).reshape(_EPAD // _CH, _CH)
    z64 = jnp.zeros((_NPAD, _D_OUT), f32)
    fb_wt = fb_w.T
    kbw_t = kan_base_w.T
    wsp_t = jnp.transpose(kan_spline_w, (2, 1, 0))

    h, st = _stage_a(xp, fb_wt, fb_b.reshape(1, -1))
    feat = _stage_b(h, st, bn_g.reshape(1, -1), bn_b.reshape(1, -1),
                    kbw_t, wsp_t)

    p1 = _sc_spmm(_D_OUT)(src_p, dst_p, feat, z64)
    m2 = _stage_c(p1, gl1_w, gl1_b.reshape(1, -1), gl2_w,
                  _D_OUT, _D_HID, _D_OUT)
    p2 = _sc_spmm(_D_OUT)(src_p, dst_p, m2, z64)
    latent = _stage_d(p2, gl2_b.reshape(1, -1), _D_OUT)
    p3 = _sc_spmm(_D_OUT)(src_p, dst_p, latent, z64)
    m4 = _stage_c(p3, dec1_w, dec1_b.reshape(1, -1), dec2_w,
                  _D_OUT, _D_HID, _D_IN)
    p4a = _sc_spmm(_D_OUT)(src_p, dst_p, m4[:, :_D_OUT], z64)
    p4b = _sc_spmm(_D_OUT)(src_p, dst_p, m4[:, _D_OUT:], z64)
    ra = _stage_d(p4a, dec2_b[:_D_OUT].reshape(1, -1), _D_OUT)
    rb = _stage_d(p4b, dec2_b[_D_OUT:].reshape(1, -1), _D_OUT)
    recon = jnp.concatenate([ra, rb], axis=1)
    return recon[:_N]


# speedup vs baseline: 9.9059x; 2.2132x over previous
import functools

import jax
import jax.numpy as jnp
import numpy as np
from jax import lax
from jax.experimental import pallas as pl
from jax.experimental.pallas import tpu as pltpu
from jax.experimental.pallas import tpu_sc as plsc

_N = 10000
_E = 320000
_D_IN = 128
_D_HID = 128
_D_OUT = 64
_GRID_SIZE = 5
_SPLINE_ORDER = 3
_NB = _GRID_SIZE + _SPLINE_ORDER

_NPAD = 10240
_EPAD = 327680
_NC, _NS, _NW = 2, 16, 32
_CH = 128
_RB = 1024
_NBLK = _NPAD // _RB

_KNOTS = [float(v) for v in
          (np.arange(-_SPLINE_ORDER, _GRID_SIZE + _SPLINE_ORDER + 1)
           * (2.0 / _GRID_SIZE) - 1.0).astype(np.float32)]


@functools.cache
def _sc_spmm(F):
    epw = _EPAD // _NW
    nch = epw // _CH
    zrows = _NPAD // _NS
    mesh = plsc.VectorSubcoreMesh(core_axis_name="c", subcore_axis_name="s")

    def body(src_hbm, dst_hbm, table_hbm, zeros_hbm, out_hbm,
             src_v, dst_v, rows_v, acc_sh, tab_sh, gsem, isem):
        cid = lax.axis_index("c")
        sid = lax.axis_index("s")
        wid = sid * _NC + cid
        iload = pltpu.async_copy(src_hbm.at[pl.ds(wid * nch, nch)], src_v, isem)
        iload2 = pltpu.async_copy(dst_hbm.at[pl.ds(wid * nch, nch)], dst_v, isem)
        pltpu.sync_copy(table_hbm.at[pl.ds(sid * zrows, zrows)],
                        tab_sh.at[pl.ds(sid * zrows, zrows)])
        pltpu.sync_copy(zeros_hbm.at[pl.ds(sid * zrows, zrows)],
                        acc_sh.at[pl.ds(sid * zrows, zrows)])
        iload.wait()
        iload2.wait()
        plsc.subcore_barrier()

        def gather(i, slot):
            pltpu.async_copy(tab_sh.at[src_v.at[i]], rows_v.at[slot], gsem)

        def wait_gather():
            pltpu.make_async_copy(tab_sh.at[src_v.at[0]],
                                  rows_v.at[0], gsem).wait()

        gather(0, 0)

        def step(i, carry):
            slot = lax.rem(i, 2)

            @pl.when(i + 1 < nch)
            def _():
                gather(i + 1, 1 - slot)

            wait_gather()
            pltpu.sync_copy(rows_v.at[slot], acc_sh.at[dst_v.at[i]], add=True)
            return carry

        lax.fori_loop(0, nch, step, 0)
        plsc.subcore_barrier()
        pltpu.sync_copy(acc_sh.at[pl.ds(sid * zrows, zrows)],
                        out_hbm.at[cid, pl.ds(sid * zrows, zrows)])

    return pl.kernel(
        body,
        out_type=jax.ShapeDtypeStruct((_NC, _NPAD, F), jnp.float32),
        mesh=mesh,
        scratch_types=[
            pltpu.VMEM((nch, _CH), jnp.int32),
            pltpu.VMEM((nch, _CH), jnp.int32),
            pltpu.VMEM((2, _CH, F), jnp.float32),
            pltpu.VMEM_SHARED((_NPAD, F), jnp.float32),
            pltpu.VMEM_SHARED((_NPAD, F), jnp.float32),
            pltpu.SemaphoreType.DMA,
            pltpu.SemaphoreType.DMA,
        ],
        compiler_params=pltpu.CompilerParams(use_tc_tiling_on_sc=False),
    )


def _full(shape):
    return pl.BlockSpec(shape, lambda i: (0,) * len(shape))


def _rows(bs, *rest):
    return pl.BlockSpec((bs,) + rest, lambda i: (i,) + (0,) * len(rest))


def _a_body(x_ref, wt_ref, b_ref, h_ref, st_ref):
    i = pl.program_id(0)
    h = jnp.dot(x_ref[...], wt_ref[...],
                preferred_element_type=jnp.float32) + b_ref[...]
    h_ref[...] = h
    rows = i * _RB + lax.broadcasted_iota(jnp.int32, (_RB, 1), 0)
    hm = jnp.where(rows < _N, h, 0.0)
    s = jnp.concatenate([jnp.sum(hm, 0, keepdims=True),
                         jnp.sum(hm * hm, 0, keepdims=True)], 0)

    @pl.when(i == 0)
    def _():
        st_ref[...] = jnp.zeros_like(st_ref)

    st_ref[...] += s


def _stage_a(xp, fb_wt, fb_b):
    return pl.pallas_call(
        _a_body,
        grid=(_NBLK,),
        in_specs=[_rows(_RB, _D_IN), _full((_D_IN, _D_HID)), _full((1, _D_HID))],
        out_specs=[_rows(_RB, _D_HID), _full((2, _D_HID))],
        out_shape=[jax.ShapeDtypeStruct((_NPAD, _D_HID), jnp.float32),
                   jax.ShapeDtypeStruct((2, _D_HID), jnp.float32)],
    )(xp, fb_wt, fb_b)


def _b_body(h_ref, st_ref, bng_ref, bnb_ref, kbwt_ref, wspt_ref, feat_ref):
    h = h_ref[...]
    mu = st_ref[0:1, :] * (1.0 / _N)
    var = st_ref[1:2, :] * (1.0 / _N) - mu * mu
    hb = bng_ref[...] * (h - mu) / jnp.sqrt(var + 1e-3) + bnb_ref[...]
    hb = jnp.where(hb > 0, hb, jnp.exp(jnp.minimum(hb, 0.0)) - 1.0)
    sil = hb / (1.0 + jnp.exp(-hb))
    bs = [jnp.where((hb >= _KNOTS[t]) & (hb < _KNOTS[t + 1]), 1.0, 0.0)
          for t in range(len(_KNOTS) - 1)]
    for k in range(1, _SPLINE_ORDER + 1):
        nxt = []
        for t in range(len(bs) - 1):
            left = (hb - _KNOTS[t]) / (_KNOTS[t + k] - _KNOTS[t])
            right = (_KNOTS[t + k + 1] - hb) / (_KNOTS[t + k + 1] - _KNOTS[t + 1])
            nxt.append(left * bs[t] + right * bs[t + 1])
        bs = nxt
    acc = jnp.dot(sil, kbwt_ref[...], preferred_element_type=jnp.float32)
    for t in range(_NB):
        acc += jnp.dot(bs[t], wspt_ref[t], preferred_element_type=jnp.float32)
    feat_ref[...] = acc


def _stage_b(h, st, bn_g, bn_b, kbw_t, wsp_t):
    return pl.pallas_call(
        _b_body,
        grid=(_NBLK,),
        in_specs=[_rows(_RB, _D_HID), _full((2, _D_HID)), _full((1, _D_HID)),
                  _full((1, _D_HID)), _full((_D_HID, _D_OUT)),
                  _full((_NB, _D_HID, _D_OUT))],
        out_specs=_rows(_RB, _D_OUT),
        out_shape=jax.ShapeDtypeStruct((_NPAD, _D_OUT), jnp.float32),
    )(h, st, bn_g, bn_b, kbw_t, wsp_t)


def _c_body(p_ref, w1_ref, b1_ref, w2_ref, out_ref):
    g = p_ref[0] + p_ref[1]
    z1 = jnp.maximum(
        jnp.dot(g, w1_ref[...], preferred_element_type=jnp.float32)
        + b1_ref[...], 0.0)
    out_ref[...] = jnp.dot(z1, w2_ref[...], preferred_element_type=jnp.float32)


def _stage_c(p, w1, b1, w2, fin, fmid, fout):
    return pl.pallas_call(
        _c_body,
        grid=(_NBLK,),
        in_specs=[pl.BlockSpec((_NC, _RB, fin), lambda i: (0, i, 0)),
                  _full((fin, fmid)), _full((1, fmid)), _full((fmid, fout))],
        out_specs=_rows(_RB, fout),
        out_shape=jax.ShapeDtypeStruct((_NPAD, fout), jnp.float32),
    )(p, w1, b1, w2)


def _d_body(p_ref, b_ref, out_ref):
    out_ref[...] = p_ref[0] + p_ref[1] + b_ref[...]


def _stage_d(p, b, f):
    return pl.pallas_call(
        _d_body,
        grid=(_NBLK,),
        in_specs=[pl.BlockSpec((_NC, _RB, f), lambda i: (0, i, 0)),
                  _full((1, f))],
        out_specs=_rows(_RB, f),
        out_shape=jax.ShapeDtypeStruct((_NPAD, f), jnp.float32),
    )(p, b)


def kernel(x, edge_index, fb_w, fb_b, bn_g, bn_b, kan_base_w, kan_spline_w,
           gl1_w, gl1_b, gl2_w, gl2_b, dec1_w, dec1_b, dec2_w, dec2_b):
    f32 = jnp.float32
    xp = jnp.zeros((_NPAD, _D_IN), f32).at[:_N].set(x)
    pad = jnp.full((_EPAD - _E,), _N, jnp.int32)
    src_p = jnp.concatenate([edge_index[0], pad]).reshape(_EPAD // _CH, _CH)
    dst_p = jnp.concatenate([edge_index[1], pad]).reshape(_EPAD // _CH, _CH)
    z64 = jnp.zeros((_NPAD, _D_OUT), f32)
    fb_wt = fb_w.T
    kbw_t = kan_base_w.T
    wsp_t = jnp.transpose(kan_spline_w, (2, 1, 0))

    h, st = _stage_a(xp, fb_wt, fb_b.reshape(1, -1))
    feat = _stage_b(h, st, bn_g.reshape(1, -1), bn_b.reshape(1, -1),
                    kbw_t, wsp_t)

    p1 = _sc_spmm(_D_OUT)(src_p, dst_p, feat, z64)
    m2 = _stage_c(p1, gl1_w, gl1_b.reshape(1, -1), gl2_w,
                  _D_OUT, _D_HID, _D_OUT)
    p2 = _sc_spmm(_D_OUT)(src_p, dst_p, m2, z64)
    latent = _stage_d(p2, gl2_b.reshape(1, -1), _D_OUT)
    p3 = _sc_spmm(_D_OUT)(src_p, dst_p, latent, z64)
    m4 = _stage_c(p3, dec1_w, dec1_b.reshape(1, -1), dec2_w,
                  _D_OUT, _D_HID, _D_IN)
    p4a = _sc_spmm(_D_OUT)(src_p, dst_p, m4[:, :_D_OUT], z64)
    p4b = _sc_spmm(_D_OUT)(src_p, dst_p, m4[:, _D_OUT:], z64)
    ra = _stage_d(p4a, dec2_b[:_D_OUT].reshape(1, -1), _D_OUT)
    rb = _stage_d(p4b, dec2_b[_D_OUT:].reshape(1, -1), _D_OUT)
    recon = jnp.concatenate([ra, rb], axis=1)
    return recon[:_N]


# speedup vs baseline: 10.2222x; 1.0319x over previous
import functools

import jax
import jax.numpy as jnp
import numpy as np
from jax import lax
from jax.experimental import pallas as pl
from jax.experimental.pallas import tpu as pltpu
from jax.experimental.pallas import tpu_sc as plsc

_N = 10000
_E = 320000
_D_IN = 128
_D_HID = 128
_D_OUT = 64
_GRID_SIZE = 5
_SPLINE_ORDER = 3
_NB = _GRID_SIZE + _SPLINE_ORDER

_NPAD = 10240
_EPAD = 327680
_NC, _NS, _NW = 2, 16, 32
_CH = 128
_RB = 1024
_NBLK = _NPAD // _RB

_KNOTS = [float(v) for v in
          (np.arange(-_SPLINE_ORDER, _GRID_SIZE + _SPLINE_ORDER + 1)
           * (2.0 / _GRID_SIZE) - 1.0).astype(np.float32)]


@functools.cache
def _sc_spmm(F):
    epw = _EPAD // _NW
    nch = epw // _CH
    zrows = _NPAD // _NS
    mesh = plsc.VectorSubcoreMesh(core_axis_name="c", subcore_axis_name="s")

    def body(src_hbm, dst_hbm, table_hbm, zeros_hbm, out_hbm,
             src_v, dst_v, rows_v, acc_sh, tab_sh, gsem, isem):
        cid = lax.axis_index("c")
        sid = lax.axis_index("s")
        wid = sid * _NC + cid
        iload = pltpu.async_copy(src_hbm.at[pl.ds(wid * nch, nch)], src_v, isem)
        iload2 = pltpu.async_copy(dst_hbm.at[pl.ds(wid * nch, nch)], dst_v, isem)
        pltpu.sync_copy(table_hbm.at[pl.ds(sid * zrows, zrows)],
                        tab_sh.at[pl.ds(sid * zrows, zrows)])
        pltpu.sync_copy(zeros_hbm.at[pl.ds(sid * zrows, zrows)],
                        acc_sh.at[pl.ds(sid * zrows, zrows)])
        iload.wait()
        iload2.wait()
        plsc.subcore_barrier()

        def gather(i, slot):
            pltpu.async_copy(tab_sh.at[src_v.at[i]], rows_v.at[slot], gsem)

        def wait_gather():
            pltpu.make_async_copy(tab_sh.at[src_v.at[0]],
                                  rows_v.at[0], gsem).wait()

        gather(0, 0)

        def step(i, carry):
            slot = lax.rem(i, 2)

            @pl.when(i + 1 < nch)
            def _():
                gather(i + 1, 1 - slot)

            wait_gather()
            pltpu.sync_copy(rows_v.at[slot], acc_sh.at[dst_v.at[i]], add=True)
            return carry

        lax.fori_loop(0, nch, step, 0)
        plsc.subcore_barrier()
        pltpu.sync_copy(acc_sh.at[pl.ds(sid * zrows, zrows)],
                        out_hbm.at[cid, pl.ds(sid * zrows, zrows)])

    return pl.kernel(
        body,
        out_type=jax.ShapeDtypeStruct((_NC, _NPAD, F), jnp.float32),
        mesh=mesh,
        scratch_types=[
            pltpu.VMEM((nch, _CH), jnp.int32),
            pltpu.VMEM((nch, _CH), jnp.int32),
            pltpu.VMEM((2, _CH, F), jnp.float32),
            pltpu.VMEM_SHARED((_NPAD, F), jnp.float32),
            pltpu.VMEM_SHARED((_NPAD, F), jnp.float32),
            pltpu.SemaphoreType.DMA,
            pltpu.SemaphoreType.DMA,
        ],
        compiler_params=pltpu.CompilerParams(use_tc_tiling_on_sc=False),
    )


def _full(shape):
    return pl.BlockSpec(shape, lambda i: (0,) * len(shape))


def _rows(bs, *rest):
    return pl.BlockSpec((bs,) + rest, lambda i: (i,) + (0,) * len(rest))


def _ab_body(x_ref, wt_ref, b_ref, bng_ref, bnb_ref, kbwt_ref, wspt_ref,
             feat_ref, h_scr, st_scr):
    p = pl.program_id(0)
    i = pl.program_id(1)

    @pl.when(p == 0)
    def _():
        h = jnp.dot(x_ref[...], wt_ref[...],
                    preferred_element_type=jnp.float32) + b_ref[...]
        h_scr[pl.ds(i * _RB, _RB), :] = h
        rows = i * _RB + lax.broadcasted_iota(jnp.int32, (_RB, 1), 0)
        hm = jnp.where(rows < _N, h, 0.0)
        s = jnp.concatenate([jnp.sum(hm, 0, keepdims=True),
                             jnp.sum(hm * hm, 0, keepdims=True)], 0)

        @pl.when(i == 0)
        def _():
            st_scr[...] = jnp.zeros_like(st_scr)

        st_scr[...] += s

    @pl.when(p == 1)
    def _():
        h = h_scr[pl.ds(i * _RB, _RB), :]
        mu = st_scr[0:1, :] * (1.0 / _N)
        var = st_scr[1:2, :] * (1.0 / _N) - mu * mu
        hb = bng_ref[...] * (h - mu) / jnp.sqrt(var + 1e-3) + bnb_ref[...]
        hb = jnp.where(hb > 0, hb, jnp.exp(jnp.minimum(hb, 0.0)) - 1.0)
        sil = hb / (1.0 + jnp.exp(-hb))
        bs = [jnp.where((hb >= _KNOTS[t]) & (hb < _KNOTS[t + 1]), 1.0, 0.0)
              for t in range(len(_KNOTS) - 1)]
        for k in range(1, _SPLINE_ORDER + 1):
            nxt = []
            for t in range(len(bs) - 1):
                left = (hb - _KNOTS[t]) / (_KNOTS[t + k] - _KNOTS[t])
                right = ((_KNOTS[t + k + 1] - hb)
                         / (_KNOTS[t + k + 1] - _KNOTS[t + 1]))
                nxt.append(left * bs[t] + right * bs[t + 1])
            bs = nxt
        acc = jnp.dot(sil, kbwt_ref[...], preferred_element_type=jnp.float32)
        for t in range(_NB):
            acc += jnp.dot(bs[t], wspt_ref[t],
                           preferred_element_type=jnp.float32)
        feat_ref[...] = acc


def _stage_ab(xp, fb_wt, fb_b, bn_g, bn_b, kbw_t, wsp_t):
    return pl.pallas_call(
        _ab_body,
        grid=(2, _NBLK),
        in_specs=[pl.BlockSpec((_RB, _D_IN), lambda p, i: (i, 0)),
                  pl.BlockSpec((_D_IN, _D_HID), lambda p, i: (0, 0)),
                  pl.BlockSpec((1, _D_HID), lambda p, i: (0, 0)),
                  pl.BlockSpec((1, _D_HID), lambda p, i: (0, 0)),
                  pl.BlockSpec((1, _D_HID), lambda p, i: (0, 0)),
                  pl.BlockSpec((_D_HID, _D_OUT), lambda p, i: (0, 0)),
                  pl.BlockSpec((_NB, _D_HID, _D_OUT), lambda p, i: (0, 0, 0))],
        out_specs=pl.BlockSpec((_RB, _D_OUT), lambda p, i: (p * i, 0)),
        out_shape=jax.ShapeDtypeStruct((_NPAD, _D_OUT), jnp.float32),
        scratch_shapes=[pltpu.VMEM((_NPAD, _D_HID), jnp.float32),
                        pltpu.VMEM((2, _D_HID), jnp.float32)],
    )(xp, fb_wt, fb_b, bn_g, bn_b, kbw_t, wsp_t)


def _c_body(p_ref, w1_ref, b1_ref, w2_ref, out_ref):
    g = p_ref[0] + p_ref[1]
    z1 = jnp.maximum(
        jnp.dot(g, w1_ref[...], preferred_element_type=jnp.float32)
        + b1_ref[...], 0.0)
    out_ref[...] = jnp.dot(z1, w2_ref[...], preferred_element_type=jnp.float32)


def _stage_c(p, w1, b1, w2, fin, fmid, fout):
    return pl.pallas_call(
        _c_body,
        grid=(_NBLK,),
        in_specs=[pl.BlockSpec((_NC, _RB, fin), lambda i: (0, i, 0)),
                  _full((fin, fmid)), _full((1, fmid)), _full((fmid, fout))],
        out_specs=_rows(_RB, fout),
        out_shape=jax.ShapeDtypeStruct((_NPAD, fout), jnp.float32),
    )(p, w1, b1, w2)


def _e_body(p_ref, w1_ref, b1_ref, w2_ref, outa_ref, outb_ref):
    g = p_ref[0] + p_ref[1]
    d1 = jnp.maximum(
        jnp.dot(g, w1_ref[...], preferred_element_type=jnp.float32)
        + b1_ref[...], 0.0)
    m4 = jnp.dot(d1, w2_ref[...], preferred_element_type=jnp.float32)
    outa_ref[...] = m4[:, :_D_OUT]
    outb_ref[...] = m4[:, _D_OUT:]


def _stage_e(p, w1, b1, w2):
    return pl.pallas_call(
        _e_body,
        grid=(_NBLK,),
        in_specs=[pl.BlockSpec((_NC, _RB, _D_OUT), lambda i: (0, i, 0)),
                  _full((_D_OUT, _D_HID)), _full((1, _D_HID)),
                  _full((_D_HID, _D_IN))],
        out_specs=[_rows(_RB, _D_OUT), _rows(_RB, _D_OUT)],
        out_shape=[jax.ShapeDtypeStruct((_NPAD, _D_OUT), jnp.float32),
                   jax.ShapeDtypeStruct((_NPAD, _D_OUT), jnp.float32)],
    )(p, w1, b1, w2)


def _f_body(pa_ref, pb_ref, b_ref, out_ref):
    ra = pa_ref[0] + pa_ref[1] + b_ref[:, :_D_OUT]
    rb = pb_ref[0] + pb_ref[1] + b_ref[:, _D_OUT:]
    out_ref[...] = jnp.concatenate([ra, rb], axis=1)


def _stage_f(pa, pb, b):
    return pl.pallas_call(
        _f_body,
        grid=(_NBLK,),
        in_specs=[pl.BlockSpec((_NC, _RB, _D_OUT), lambda i: (0, i, 0)),
                  pl.BlockSpec((_NC, _RB, _D_OUT), lambda i: (0, i, 0)),
                  _full((1, _D_IN))],
        out_specs=_rows(_RB, _D_IN),
        out_shape=jax.ShapeDtypeStruct((_N, _D_IN), jnp.float32),
    )(pa, pb, b)


def _d_body(p_ref, b_ref, out_ref):
    out_ref[...] = p_ref[0] + p_ref[1] + b_ref[...]


def _stage_d(p, b, f):
    return pl.pallas_call(
        _d_body,
        grid=(_NBLK,),
        in_specs=[pl.BlockSpec((_NC, _RB, f), lambda i: (0, i, 0)),
                  _full((1, f))],
        out_specs=_rows(_RB, f),
        out_shape=jax.ShapeDtypeStruct((_NPAD, f), jnp.float32),
    )(p, b)


def kernel(x, edge_index, fb_w, fb_b, bn_g, bn_b, kan_base_w, kan_spline_w,
           gl1_w, gl1_b, gl2_w, gl2_b, dec1_w, dec1_b, dec2_w, dec2_b):
    f32 = jnp.float32
    xp = jnp.zeros((_NPAD, _D_IN), f32).at[:_N].set(x)
    pad = jnp.full((_EPAD - _E,), _N, jnp.int32)
    src_p = jnp.concatenate([edge_index[0], pad]).reshape(_EPAD // _CH, _CH)
    dst_p = jnp.concatenate([edge_index[1], pad]).reshape(_EPAD // _CH, _CH)
    z64 = jnp.zeros((_NPAD, _D_OUT), f32)
    fb_wt = fb_w.T
    kbw_t = kan_base_w.T
    wsp_t = jnp.transpose(kan_spline_w, (2, 1, 0))

    feat = _stage_ab(xp, fb_wt, fb_b.reshape(1, -1), bn_g.reshape(1, -1),
                     bn_b.reshape(1, -1), kbw_t, wsp_t)

    p1 = _sc_spmm(_D_OUT)(src_p, dst_p, feat, z64)
    m2 = _stage_c(p1, gl1_w, gl1_b.reshape(1, -1), gl2_w,
                  _D_OUT, _D_HID, _D_OUT)
    p2 = _sc_spmm(_D_OUT)(src_p, dst_p, m2, z64)
    latent = _stage_d(p2, gl2_b.reshape(1, -1), _D_OUT)
    p3 = _sc_spmm(_D_OUT)(src_p, dst_p, latent, z64)
    m4a, m4b = _stage_e(p3, dec1_w, dec1_b.reshape(1, -1), dec2_w)
    p4a = _sc_spmm(_D_OUT)(src_p, dst_p, m4a, z64)
    p4b = _sc_spmm(_D_OUT)(src_p, dst_p, m4b, z64)
    return _stage_f(p4a, p4b, dec2_b.reshape(1, -1))


# speedup vs baseline: 10.6291x; 1.0398x over previous
import functools

import jax
import jax.numpy as jnp
import numpy as np
from jax import lax
from jax.experimental import pallas as pl
from jax.experimental.pallas import tpu as pltpu
from jax.experimental.pallas import tpu_sc as plsc

_N = 10000
_E = 320000
_D_IN = 128
_D_HID = 128
_D_OUT = 64
_GRID_SIZE = 5
_SPLINE_ORDER = 3
_NB = _GRID_SIZE + _SPLINE_ORDER

_NPAD = 10240
_NC, _NS, _NW = 2, 16, 32
_CH = 80
_RB = 1024
_NBLK = _NPAD // _RB

_KNOTS = [float(v) for v in
          (np.arange(-_SPLINE_ORDER, _GRID_SIZE + _SPLINE_ORDER + 1)
           * (2.0 / _GRID_SIZE) - 1.0).astype(np.float32)]


@functools.cache
def _sc_spmm(F):
    nch = _E // _NW // _CH
    zrows = _NPAD // _NS
    mesh = plsc.VectorSubcoreMesh(core_axis_name="c", subcore_axis_name="s")

    def body(src_hbm, dst_hbm, table_hbm, zeros_hbm, out_hbm,
             src_v, dst_v, rows_v, acc_sh, tab_sh, gsem, isem):
        cid = lax.axis_index("c")
        sid = lax.axis_index("s")
        wid = sid * _NC + cid
        iload = pltpu.async_copy(src_hbm.at[pl.ds(wid * nch, nch)], src_v, isem)
        iload2 = pltpu.async_copy(dst_hbm.at[pl.ds(wid * nch, nch)], dst_v, isem)
        pltpu.sync_copy(table_hbm.at[pl.ds(sid * zrows, zrows)],
                        tab_sh.at[pl.ds(sid * zrows, zrows)])
        pltpu.sync_copy(zeros_hbm.at[pl.ds(sid * zrows, zrows)],
                        acc_sh.at[pl.ds(sid * zrows, zrows)])
        iload.wait()
        iload2.wait()
        plsc.subcore_barrier()

        def gather(i, slot):
            pltpu.async_copy(tab_sh.at[src_v.at[i]], rows_v.at[slot], gsem)

        def wait_gather():
            pltpu.make_async_copy(tab_sh.at[src_v.at[0]],
                                  rows_v.at[0], gsem).wait()

        gather(0, 0)

        def step(i, carry):
            slot = lax.rem(i, 2)

            @pl.when(i + 1 < nch)
            def _():
                gather(i + 1, 1 - slot)

            wait_gather()
            pltpu.sync_copy(rows_v.at[slot], acc_sh.at[dst_v.at[i]], add=True)
            return carry

        lax.fori_loop(0, nch, step, 0)
        plsc.subcore_barrier()
        pltpu.sync_copy(acc_sh.at[pl.ds(sid * zrows, zrows)],
                        out_hbm.at[cid, pl.ds(sid * zrows, zrows)])

    return pl.kernel(
        body,
        out_type=jax.ShapeDtypeStruct((_NC, _NPAD, F), jnp.float32),
        mesh=mesh,
        scratch_types=[
            pltpu.VMEM((nch, _CH), jnp.int32),
            pltpu.VMEM((nch, _CH), jnp.int32),
            pltpu.VMEM((2, _CH, F), jnp.float32),
            pltpu.VMEM_SHARED((_NPAD, F), jnp.float32),
            pltpu.VMEM_SHARED((_NPAD, F), jnp.float32),
            pltpu.SemaphoreType.DMA,
            pltpu.SemaphoreType.DMA,
        ],
        compiler_params=pltpu.CompilerParams(use_tc_tiling_on_sc=False),
    )


def _full(shape):
    return pl.BlockSpec(shape, lambda i: (0,) * len(shape))


def _rows(bs, *rest):
    return pl.BlockSpec((bs,) + rest, lambda i: (i,) + (0,) * len(rest))


def _ab_body(x_ref, wt_ref, b_ref, bng_ref, bnb_ref, kbwt_ref, wspt_ref,
             feat_ref, h_scr, st_scr):
    p = pl.program_id(0)
    i = pl.program_id(1)

    @pl.when(p == 0)
    def _():
        h = jnp.dot(x_ref[...], wt_ref[...],
                    preferred_element_type=jnp.float32) + b_ref[...]
        h_scr[pl.ds(i * _RB, _RB), :] = h
        rows = i * _RB + lax.broadcasted_iota(jnp.int32, (_RB, 1), 0)
        hm = jnp.where(rows < _N, h, 0.0)
        s = jnp.concatenate([jnp.sum(hm, 0, keepdims=True),
                             jnp.sum(hm * hm, 0, keepdims=True)], 0)

        @pl.when(i == 0)
        def _():
            st_scr[...] = jnp.zeros_like(st_scr)

        st_scr[...] += s

    @pl.when(p == 1)
    def _():
        h = h_scr[pl.ds(i * _RB, _RB), :]
        mu = st_scr[0:1, :] * (1.0 / _N)
        var = st_scr[1:2, :] * (1.0 / _N) - mu * mu
        hb = bng_ref[...] * (h - mu) / jnp.sqrt(var + 1e-3) + bnb_ref[...]
        hb = jnp.where(hb > 0, hb, jnp.exp(jnp.minimum(hb, 0.0)) - 1.0)
        sil = hb / (1.0 + jnp.exp(-hb))
        bs = [jnp.where((hb >= _KNOTS[t]) & (hb < _KNOTS[t + 1]), 1.0, 0.0)
              for t in range(len(_KNOTS) - 1)]
        h_knot = 2.0 / _GRID_SIZE
        for k in range(1, _SPLINE_ORDER + 1):
            hbk = hb * (1.0 / (k * h_knot))
            nxt = []
            for t in range(len(bs) - 1):
                lc = _KNOTS[t] / (k * h_knot)
                rc = _KNOTS[t + k + 1] / (k * h_knot)
                nxt.append((hbk - lc) * bs[t] + (rc - hbk) * bs[t + 1])
            bs = nxt
        acc = jnp.dot(sil, kbwt_ref[...], preferred_element_type=jnp.float32)
        for t in range(_NB):
            acc += jnp.dot(bs[t], wspt_ref[t],
                           preferred_element_type=jnp.float32)
        feat_ref[...] = acc


def _stage_ab(xp, fb_wt, fb_b, bn_g, bn_b, kbw_t, wsp_t):
    return pl.pallas_call(
        _ab_body,
        grid=(2, _NBLK),
        in_specs=[pl.BlockSpec((_RB, _D_IN), lambda p, i: (i, 0)),
                  pl.BlockSpec((_D_IN, _D_HID), lambda p, i: (0, 0)),
                  pl.BlockSpec((1, _D_HID), lambda p, i: (0, 0)),
                  pl.BlockSpec((1, _D_HID), lambda p, i: (0, 0)),
                  pl.BlockSpec((1, _D_HID), lambda p, i: (0, 0)),
                  pl.BlockSpec((_D_HID, _D_OUT), lambda p, i: (0, 0)),
                  pl.BlockSpec((_NB, _D_HID, _D_OUT), lambda p, i: (0, 0, 0))],
        out_specs=pl.BlockSpec((_RB, _D_OUT), lambda p, i: (p * i, 0)),
        out_shape=jax.ShapeDtypeStruct((_NPAD, _D_OUT), jnp.float32),
        scratch_shapes=[pltpu.VMEM((_NPAD, _D_HID), jnp.float32),
                        pltpu.VMEM((2, _D_HID), jnp.float32)],
    )(xp, fb_wt, fb_b, bn_g, bn_b, kbw_t, wsp_t)


def _c_body(p_ref, w1_ref, b1_ref, w2_ref, out_ref):
    g = p_ref[0] + p_ref[1]
    z1 = jnp.maximum(
        jnp.dot(g, w1_ref[...], preferred_element_type=jnp.float32)
        + b1_ref[...], 0.0)
    out_ref[...] = jnp.dot(z1, w2_ref[...], preferred_element_type=jnp.float32)


def _stage_c(p, w1, b1, w2, fin, fmid, fout):
    return pl.pallas_call(
        _c_body,
        grid=(_NBLK,),
        in_specs=[pl.BlockSpec((_NC, _RB, fin), lambda i: (0, i, 0)),
                  _full((fin, fmid)), _full((1, fmid)), _full((fmid, fout))],
        out_specs=_rows(_RB, fout),
        out_shape=jax.ShapeDtypeStruct((_NPAD, fout), jnp.float32),
    )(p, w1, b1, w2)


def _e_body(p_ref, w1_ref, b1_ref, w2_ref, outa_ref, outb_ref):
    g = p_ref[0] + p_ref[1]
    d1 = jnp.maximum(
        jnp.dot(g, w1_ref[...], preferred_element_type=jnp.float32)
        + b1_ref[...], 0.0)
    m4 = jnp.dot(d1, w2_ref[...], preferred_element_type=jnp.float32)
    outa_ref[...] = m4[:, :_D_OUT]
    outb_ref[...] = m4[:, _D_OUT:]


def _stage_e(p, w1, b1, w2):
    return pl.pallas_call(
        _e_body,
        grid=(_NBLK,),
        in_specs=[pl.BlockSpec((_NC, _RB, _D_OUT), lambda i: (0, i, 0)),
                  _full((_D_OUT, _D_HID)), _full((1, _D_HID)),
                  _full((_D_HID, _D_IN))],
        out_specs=[_rows(_RB, _D_OUT), _rows(_RB, _D_OUT)],
        out_shape=[jax.ShapeDtypeStruct((_NPAD, _D_OUT), jnp.float32),
                   jax.ShapeDtypeStruct((_NPAD, _D_OUT), jnp.float32)],
    )(p, w1, b1, w2)


def _f_body(pa_ref, pb_ref, b_ref, out_ref):
    ra = pa_ref[0] + pa_ref[1] + b_ref[:, :_D_OUT]
    rb = pb_ref[0] + pb_ref[1] + b_ref[:, _D_OUT:]
    out_ref[...] = jnp.concatenate([ra, rb], axis=1)


def _stage_f(pa, pb, b):
    return pl.pallas_call(
        _f_body,
        grid=(_NBLK,),
        in_specs=[pl.BlockSpec((_NC, _RB, _D_OUT), lambda i: (0, i, 0)),
                  pl.BlockSpec((_NC, _RB, _D_OUT), lambda i: (0, i, 0)),
                  _full((1, _D_IN))],
        out_specs=_rows(_RB, _D_IN),
        out_shape=jax.ShapeDtypeStruct((_N, _D_IN), jnp.float32),
    )(pa, pb, b)


def _d_body(p_ref, b_ref, out_ref):
    out_ref[...] = p_ref[0] + p_ref[1] + b_ref[...]


def _stage_d(p, b, f):
    return pl.pallas_call(
        _d_body,
        grid=(_NBLK,),
        in_specs=[pl.BlockSpec((_NC, _RB, f), lambda i: (0, i, 0)),
                  _full((1, f))],
        out_specs=_rows(_RB, f),
        out_shape=jax.ShapeDtypeStruct((_NPAD, f), jnp.float32),
    )(p, b)


def kernel(x, edge_index, fb_w, fb_b, bn_g, bn_b, kan_base_w, kan_spline_w,
           gl1_w, gl1_b, gl2_w, gl2_b, dec1_w, dec1_b, dec2_w, dec2_b):
    f32 = jnp.float32
    xp = jnp.zeros((_NPAD, _D_IN), f32).at[:_N].set(x)
    src_p = edge_index[0].reshape(_E // _CH, _CH)
    dst_p = edge_index[1].reshape(_E // _CH, _CH)
    z64 = jnp.zeros((_NPAD, _D_OUT), f32)
    fb_wt = fb_w.T
    kbw_t = kan_base_w.T
    wsp_t = jnp.transpose(kan_spline_w, (2, 1, 0))

    feat = _stage_ab(xp, fb_wt, fb_b.reshape(1, -1), bn_g.reshape(1, -1),
                     bn_b.reshape(1, -1), kbw_t, wsp_t)

    p1 = _sc_spmm(_D_OUT)(src_p, dst_p, feat, z64)
    m2 = _stage_c(p1, gl1_w, gl1_b.reshape(1, -1), gl2_w,
                  _D_OUT, _D_HID, _D_OUT)
    p2 = _sc_spmm(_D_OUT)(src_p, dst_p, m2, z64)
    latent = _stage_d(p2, gl2_b.reshape(1, -1), _D_OUT)
    p3 = _sc_spmm(_D_OUT)(src_p, dst_p, latent, z64)
    m4a, m4b = _stage_e(p3, dec1_w, dec1_b.reshape(1, -1), dec2_w)
    p4a = _sc_spmm(_D_OUT)(src_p, dst_p, m4a, z64)
    p4b = _sc_spmm(_D_OUT)(src_p, dst_p, m4b, z64)
    return _stage_f(p4a, p4b, dec2_b.reshape(1, -1))


# speedup vs baseline: 11.0885x; 1.0432x over previous
import functools

import jax
import jax.numpy as jnp
import numpy as np
from jax import lax
from jax.experimental import pallas as pl
from jax.experimental.pallas import tpu as pltpu
from jax.experimental.pallas import tpu_sc as plsc

_N = 10000
_E = 320000
_D_IN = 128
_D_HID = 128
_D_OUT = 64
_GRID_SIZE = 5
_SPLINE_ORDER = 3
_NB = _GRID_SIZE + _SPLINE_ORDER

_NPAD = 10240
_NC, _NS, _NW = 2, 16, 32
_CH = 80
_RB = 1024
_NBLK = _NPAD // _RB

_KNOTS = [float(v) for v in
          (np.arange(-_SPLINE_ORDER, _GRID_SIZE + _SPLINE_ORDER + 1)
           * (2.0 / _GRID_SIZE) - 1.0).astype(np.float32)]


@functools.cache
def _sc_spmm(F):
    nch = _E // _NW // _CH
    zrows = _NPAD // _NS
    mesh = plsc.VectorSubcoreMesh(core_axis_name="c", subcore_axis_name="s")

    def body(edge_hbm, table_hbm, zeros_hbm, out_hbm,
             src_v, dst_v, rows_v, acc_sh, tab_sh, gsem, isem):
        cid = lax.axis_index("c")
        sid = lax.axis_index("s")
        wid = sid * _NC + cid
        iload = pltpu.async_copy(edge_hbm.at[0, pl.ds(wid * nch, nch)],
                                 src_v, isem)
        iload2 = pltpu.async_copy(edge_hbm.at[1, pl.ds(wid * nch, nch)],
                                  dst_v, isem)
        pltpu.sync_copy(table_hbm.at[pl.ds(sid * zrows, zrows)],
                        tab_sh.at[pl.ds(sid * zrows, zrows)])
        pltpu.sync_copy(zeros_hbm.at[pl.ds(sid * zrows, zrows)],
                        acc_sh.at[pl.ds(sid * zrows, zrows)])
        iload.wait()
        iload2.wait()
        plsc.subcore_barrier()

        def gather(i, slot):
            pltpu.async_copy(tab_sh.at[src_v.at[i]], rows_v.at[slot], gsem)

        def wait_gather():
            pltpu.make_async_copy(tab_sh.at[src_v.at[0]],
                                  rows_v.at[0], gsem).wait()

        gather(0, 0)

        def step(i, carry):
            slot = lax.rem(i, 2)

            @pl.when(i + 1 < nch)
            def _():
                gather(i + 1, 1 - slot)

            wait_gather()
            pltpu.sync_copy(rows_v.at[slot], acc_sh.at[dst_v.at[i]], add=True)
            return carry

        lax.fori_loop(0, nch, step, 0)
        plsc.subcore_barrier()
        pltpu.sync_copy(acc_sh.at[pl.ds(sid * zrows, zrows)],
                        out_hbm.at[cid, pl.ds(sid * zrows, zrows)])

    return pl.kernel(
        body,
        out_type=jax.ShapeDtypeStruct((_NC, _NPAD, F), jnp.float32),
        mesh=mesh,
        scratch_types=[
            pltpu.VMEM((nch, _CH), jnp.int32),
            pltpu.VMEM((nch, _CH), jnp.int32),
            pltpu.VMEM((2, _CH, F), jnp.float32),
            pltpu.VMEM_SHARED((_NPAD, F), jnp.float32),
            pltpu.VMEM_SHARED((_NPAD, F), jnp.float32),
            pltpu.SemaphoreType.DMA,
            pltpu.SemaphoreType.DMA,
        ],
        compiler_params=pltpu.CompilerParams(use_tc_tiling_on_sc=False),
    )


def _full(shape):
    return pl.BlockSpec(shape, lambda i: (0,) * len(shape))


def _rows(bs, *rest):
    return pl.BlockSpec((bs,) + rest, lambda i: (i,) + (0,) * len(rest))


def _ab_body(x_ref, wt_ref, b_ref, bng_ref, bnb_ref, kbwt_ref, wspt_ref,
             feat_ref, h_scr, st_scr):
    p = pl.program_id(0)
    i = pl.program_id(1)

    @pl.when(p == 0)
    def _():
        h = jnp.dot(x_ref[...], wt_ref[...],
                    preferred_element_type=jnp.float32) + b_ref[...]
        h_scr[pl.ds(i * _RB, _RB), :] = h
        rows = i * _RB + lax.broadcasted_iota(jnp.int32, (_RB, 1), 0)
        hm = jnp.where(rows < _N, h, 0.0)
        s = jnp.concatenate([jnp.sum(hm, 0, keepdims=True),
                             jnp.sum(hm * hm, 0, keepdims=True)], 0)

        @pl.when(i == 0)
        def _():
            st_scr[...] = jnp.zeros_like(st_scr)

        st_scr[...] += s

    @pl.when(p == 1)
    def _():
        h = h_scr[pl.ds(i * _RB, _RB), :]
        mu = st_scr[0:1, :] * (1.0 / _N)
        var = st_scr[1:2, :] * (1.0 / _N) - mu * mu
        hb = bng_ref[...] * (h - mu) / jnp.sqrt(var + 1e-3) + bnb_ref[...]
        hb = jnp.where(hb > 0, hb, jnp.exp(jnp.minimum(hb, 0.0)) - 1.0)
        sil = hb / (1.0 + jnp.exp(-hb))
        h_knot = 2.0 / _GRID_SIZE
        u = (hb - _KNOTS[0]) * (1.0 / h_knot)
        c = jnp.floor(u)
        f = u - c
        f2 = f * f
        f3 = f2 * f
        g = 1.0 - f
        p0 = f3 * (1.0 / 6.0)
        p1 = (-3.0 * f3 + 3.0 * f2 + 3.0 * f + 1.0) * (1.0 / 6.0)
        p2 = (3.0 * f3 - 6.0 * f2 + 4.0) * (1.0 / 6.0)
        p3 = g * g * g * (1.0 / 6.0)
        ps = [p0, p1, p2, p3]
        zero = jnp.zeros_like(hb)
        bs = []
        for t in range(_NB):
            b = zero
            for j in range(4):
                b = jnp.where(c == float(t + j), ps[j], b)
            bs.append(b)
        acc = jnp.dot(sil, kbwt_ref[...], preferred_element_type=jnp.float32)
        for t in range(_NB):
            acc += jnp.dot(bs[t], wspt_ref[t],
                           preferred_element_type=jnp.float32)
        feat_ref[...] = acc


def _stage_ab(xp, fb_wt, fb_b, bn_g, bn_b, kbw_t, wsp_t):
    return pl.pallas_call(
        _ab_body,
        grid=(2, _NBLK),
        in_specs=[pl.BlockSpec((_RB, _D_IN), lambda p, i: (i, 0)),
                  pl.BlockSpec((_D_IN, _D_HID), lambda p, i: (0, 0)),
                  pl.BlockSpec((1, _D_HID), lambda p, i: (0, 0)),
                  pl.BlockSpec((1, _D_HID), lambda p, i: (0, 0)),
                  pl.BlockSpec((1, _D_HID), lambda p, i: (0, 0)),
                  pl.BlockSpec((_D_HID, _D_OUT), lambda p, i: (0, 0)),
                  pl.BlockSpec((_NB, _D_HID, _D_OUT), lambda p, i: (0, 0, 0))],
        out_specs=pl.BlockSpec((_RB, _D_OUT), lambda p, i: (p * i, 0)),
        out_shape=jax.ShapeDtypeStruct((_NPAD, _D_OUT), jnp.float32),
        scratch_shapes=[pltpu.VMEM((_NPAD, _D_HID), jnp.float32),
                        pltpu.VMEM((2, _D_HID), jnp.float32)],
    )(xp, fb_wt, fb_b, bn_g, bn_b, kbw_t, wsp_t)


def _c_body(p_ref, w1_ref, b1_ref, w2_ref, out_ref):
    g = p_ref[0] + p_ref[1]
    z1 = jnp.maximum(
        jnp.dot(g, w1_ref[...], preferred_element_type=jnp.float32)
        + b1_ref[...], 0.0)
    out_ref[...] = jnp.dot(z1, w2_ref[...], preferred_element_type=jnp.float32)


def _stage_c(p, w1, b1, w2, fin, fmid, fout):
    return pl.pallas_call(
        _c_body,
        grid=(_NBLK,),
        in_specs=[pl.BlockSpec((_NC, _RB, fin), lambda i: (0, i, 0)),
                  _full((fin, fmid)), _full((1, fmid)), _full((fmid, fout))],
        out_specs=_rows(_RB, fout),
        out_shape=jax.ShapeDtypeStruct((_NPAD, fout), jnp.float32),
    )(p, w1, b1, w2)


def _e_body(p_ref, w1_ref, b1_ref, w2_ref, outa_ref, outb_ref):
    g = p_ref[0] + p_ref[1]
    d1 = jnp.maximum(
        jnp.dot(g, w1_ref[...], preferred_element_type=jnp.float32)
        + b1_ref[...], 0.0)
    m4 = jnp.dot(d1, w2_ref[...], preferred_element_type=jnp.float32)
    outa_ref[...] = m4[:, :_D_OUT]
    outb_ref[...] = m4[:, _D_OUT:]


def _stage_e(p, w1, b1, w2):
    return pl.pallas_call(
        _e_body,
        grid=(_NBLK,),
        in_specs=[pl.BlockSpec((_NC, _RB, _D_OUT), lambda i: (0, i, 0)),
                  _full((_D_OUT, _D_HID)), _full((1, _D_HID)),
                  _full((_D_HID, _D_IN))],
        out_specs=[_rows(_RB, _D_OUT), _rows(_RB, _D_OUT)],
        out_shape=[jax.ShapeDtypeStruct((_NPAD, _D_OUT), jnp.float32),
                   jax.ShapeDtypeStruct((_NPAD, _D_OUT), jnp.float32)],
    )(p, w1, b1, w2)


def _f_body(pa_ref, pb_ref, b_ref, out_ref):
    ra = pa_ref[0] + pa_ref[1] + b_ref[:, :_D_OUT]
    rb = pb_ref[0] + pb_ref[1] + b_ref[:, _D_OUT:]
    out_ref[...] = jnp.concatenate([ra, rb], axis=1)


def _stage_f(pa, pb, b):
    return pl.pallas_call(
        _f_body,
        grid=(_NBLK,),
        in_specs=[pl.BlockSpec((_NC, _RB, _D_OUT), lambda i: (0, i, 0)),
                  pl.BlockSpec((_NC, _RB, _D_OUT), lambda i: (0, i, 0)),
                  _full((1, _D_IN))],
        out_specs=_rows(_RB, _D_IN),
        out_shape=jax.ShapeDtypeStruct((_N, _D_IN), jnp.float32),
    )(pa, pb, b)


def _d_body(p_ref, b_ref, out_ref):
    out_ref[...] = p_ref[0] + p_ref[1] + b_ref[...]


def _stage_d(p, b, f):
    return pl.pallas_call(
        _d_body,
        grid=(_NBLK,),
        in_specs=[pl.BlockSpec((_NC, _RB, f), lambda i: (0, i, 0)),
                  _full((1, f))],
        out_specs=_rows(_RB, f),
        out_shape=jax.ShapeDtypeStruct((_NPAD, f), jnp.float32),
    )(p, b)


def kernel(x, edge_index, fb_w, fb_b, bn_g, bn_b, kan_base_w, kan_spline_w,
           gl1_w, gl1_b, gl2_w, gl2_b, dec1_w, dec1_b, dec2_w, dec2_b):
    f32 = jnp.float32
    xp = jnp.zeros((_NPAD, _D_IN), f32).at[:_N].set(x)
    edges = edge_index.reshape(2, _E // _CH, _CH)
    z64 = jnp.zeros((_NPAD, _D_OUT), f32)
    fb_wt = fb_w.T
    kbw_t = kan_base_w.T
    wsp_t = jnp.transpose(kan_spline_w, (2, 1, 0))

    feat = _stage_ab(xp, fb_wt, fb_b.reshape(1, -1), bn_g.reshape(1, -1),
                     bn_b.reshape(1, -1), kbw_t, wsp_t)

    p1 = _sc_spmm(_D_OUT)(edges, feat, z64)
    m2 = _stage_c(p1, gl1_w, gl1_b.reshape(1, -1), gl2_w,
                  _D_OUT, _D_HID, _D_OUT)
    p2 = _sc_spmm(_D_OUT)(edges, m2, z64)
    latent = _stage_d(p2, gl2_b.reshape(1, -1), _D_OUT)
    p3 = _sc_spmm(_D_OUT)(edges, latent, z64)
    m4a, m4b = _stage_e(p3, dec1_w, dec1_b.reshape(1, -1), dec2_w)
    p4a = _sc_spmm(_D_OUT)(edges, m4a, z64)
    p4b = _sc_spmm(_D_OUT)(edges, m4b, z64)
    return _stage_f(p4a, p4b, dec2_b.reshape(1, -1))
